# baseline (speedup 1.0000x reference)
import jax
import jax.numpy as jnp
from jax import lax
from jax.experimental import pallas as pl
from jax.experimental.pallas import tpu as pltpu

_EOS = 2
_B = 128
_T = 4096
_V = 100000
_ROWS = 32


def _body(ids_ref, x_ref, o_ref):
    ids = ids_ref[...]
    eos_count = jnp.sum((ids == _EOS).astype(jnp.int32), axis=1)
    eos_count_init = eos_count
    done = (eos_count - eos_count_init) >= 2
    x = x_ref[...]
    col = jax.lax.broadcasted_iota(jnp.int32, x.shape, 1)
    masked = jnp.where(col == _EOS, 0.0, float("-inf"))
    o_ref[...] = jnp.where(done[:, None], masked, x)


def kernel(input_ids, scores):
    grid = (_B // _ROWS,)
    return pl.pallas_call(
        _body,
        grid=grid,
        in_specs=[
            pl.BlockSpec((_ROWS, _T), lambda i: (i, 0)),
            pl.BlockSpec((_ROWS, _V), lambda i: (i, 0)),
        ],
        out_specs=pl.BlockSpec((_ROWS, _V), lambda i: (i, 0)),
        out_shape=jax.ShapeDtypeStruct((_B, _V), jnp.float32),
        compiler_params=pltpu.CompilerParams(
            dimension_semantics=("arbitrary",),
            allow_input_fusion=[False, True],
        ),
    )(input_ids.astype(jnp.int32), scores * lax.optimization_barrier(jnp.float32(1.0)))


# speedup vs baseline: 1.3456x; 1.3456x over previous
import jax
import jax.numpy as jnp
from jax.experimental import pallas as pl
from jax.experimental.pallas import tpu as pltpu

_EOS = 2
_B = 128
_T = 4096
_V = 100000
_ROWS = 32


def _body(ids_ref, x_ref, o_ref):
    ids = ids_ref[...]
    eos_count = jnp.sum((ids == _EOS).astype(jnp.int32), axis=1)
    eos_count_init = eos_count
    done = (eos_count - eos_count_init) >= 2
    x = x_ref[...]
    col = jax.lax.broadcasted_iota(jnp.int32, x.shape, 1)
    masked = jnp.where(col == _EOS, 0.0, float("-inf"))
    o_ref[...] = jnp.where(done[:, None], masked, x)


def kernel(input_ids, scores):
    grid = (_B // _ROWS,)
    return pl.pallas_call(
        _body,
        grid=grid,
        in_specs=[
            pl.BlockSpec((_ROWS, _T), lambda i: (i, 0)),
            pl.BlockSpec((_ROWS, _V), lambda i: (i, 0)),
        ],
        out_specs=pl.BlockSpec((_ROWS, _V), lambda i: (i, 0)),
        out_shape=jax.ShapeDtypeStruct((_B, _V), jnp.float32),
        compiler_params=pltpu.CompilerParams(
            dimension_semantics=("arbitrary",),
        ),
    )(input_ids.astype(jnp.int32), scores)
